# baseline (speedup 1.0000x reference)
import jax
import jax.numpy as jnp
import numpy as np
from jax import lax
from jax.experimental import pallas as pl
from jax.experimental.pallas import tpu as pltpu
from jax.experimental.pallas import tpu_sc as plsc

_NUM_CLASSES = 1000
_SMOOTHING = 0.1
_BATCH = 16384

_NUM_WORKERS = 32
_ROWS = 32
_COLW = 512
_NCHUNKS = _BATCH // _COLW
_NBUF = 6
_LANES = 16
_GROUPS = _COLW // _LANES

_BASE = float(np.float32(_SMOOTHING / _NUM_CLASSES))
_PEAK = float(np.float32(np.float32(_BASE) + np.float32(1.0 - _SMOOTHING)))


def _sc_body(target_hbm, out_hbm, tgt_v, *rest):
    bufs = rest[:_NBUF]
    sems = rest[_NBUF:2 * _NBUF]
    tgt_sem = rest[2 * _NBUF]
    wid = lax.axis_index("s") * 2 + lax.axis_index("c")
    r0 = jnp.minimum(wid * _ROWS, _NUM_CLASSES - _ROWS)

    base_vec = jnp.full((_LANES,), _BASE, jnp.float32)
    peak_vec = jnp.full((_LANES,), _PEAK, jnp.float32)
    lane_iota = lax.broadcasted_iota(jnp.int32, (_LANES,), 0)

    tgt_cp = pltpu.async_copy(target_hbm, tgt_v, tgt_sem)

    def fill_row(r, _):
        def fill_grp(g, _):
            for b in bufs:
                b[r, pl.ds(g * _LANES, _LANES)] = base_vec
            return 0
        lax.fori_loop(0, _GROUPS, fill_grp, 0, unroll=4)
        return 0

    lax.fori_loop(0, _ROWS, fill_row, 0)
    tgt_cp.wait()

    def scatter_chunk(c, buf, value_vec):
        def grp(g, _):
            t = tgt_v[pl.ds(c * _COLW + g * _LANES, _LANES)]
            rows = t - r0
            mask = (t >= r0) & (t < r0 + _ROWS)
            plsc.store_scatter(buf, [rows, lane_iota + g * _LANES], value_vec,
                               mask=mask)
            return 0
        lax.fori_loop(0, _GROUPS, grp, 0, unroll=4)

    copies = [None] * _NBUF
    for c in range(_NCHUNKS):
        slot = c % _NBUF
        buf = bufs[slot]
        if copies[slot] is not None:
            copies[slot].wait()
            scatter_chunk(c - _NBUF, buf, base_vec)
        scatter_chunk(c, buf, peak_vec)
        copies[slot] = pltpu.async_copy(
            buf, out_hbm.at[pl.ds(r0, _ROWS), pl.ds(c * _COLW, _COLW)],
            sems[slot])

    for slot in range(_NBUF):
        copies[slot].wait()


@jax.jit
def _sc_call(target):
    mesh = plsc.VectorSubcoreMesh(core_axis_name="c", subcore_axis_name="s")
    q_t = pl.kernel(
        _sc_body,
        mesh=mesh,
        compiler_params=pltpu.CompilerParams(needs_layout_passes=False),
        out_type=jax.ShapeDtypeStruct((_NUM_CLASSES, _BATCH), jnp.float32),
        scratch_types=[
            pltpu.VMEM((_BATCH,), jnp.int32),
        ] + [pltpu.VMEM((_ROWS, _COLW), jnp.float32)] * _NBUF
          + [pltpu.SemaphoreType.DMA] * (_NBUF + 1),
    )(target)
    return q_t.T


def kernel(target, pred):
    del pred
    return _sc_call(target)


# speedup vs baseline: 1.0794x; 1.0794x over previous
import jax
import jax.numpy as jnp
import numpy as np
from jax import lax
from jax.experimental import pallas as pl
from jax.experimental.pallas import tpu as pltpu
from jax.experimental.pallas import tpu_sc as plsc

_NUM_CLASSES = 1000
_SMOOTHING = 0.1
_BATCH = 16384

_NUM_WORKERS = 32
_ROWS = 32
_COLW = 512
_NCHUNKS = _BATCH // _COLW
_NBUF = 6
_LANES = 16
_GROUPS = _COLW // _LANES

_BASE = float(np.float32(_SMOOTHING / _NUM_CLASSES))
_PEAK = float(np.float32(np.float32(_BASE) + np.float32(1.0 - _SMOOTHING)))


def _sc_body(target_hbm, out_hbm, tgt_v, *rest):
    bufs = rest[:_NBUF]
    sems = rest[_NBUF:2 * _NBUF]
    tgt_sem = rest[2 * _NBUF]
    wid = lax.axis_index("s") * 2 + lax.axis_index("c")
    r0 = jnp.minimum(wid * _ROWS, _NUM_CLASSES - _ROWS)

    base_vec = jnp.full((_LANES,), _BASE, jnp.float32)
    peak_vec = jnp.full((_LANES,), _PEAK, jnp.float32)
    lane_iota = lax.broadcasted_iota(jnp.int32, (_LANES,), 0)

    tgt_cp = pltpu.async_copy(target_hbm, tgt_v, tgt_sem)

    def fill_row(r, _):
        def fill_grp(g, _):
            for b in bufs:
                b[r, pl.ds(g * _LANES, _LANES)] = base_vec
            return 0
        lax.fori_loop(0, _GROUPS, fill_grp, 0)
        return 0

    lax.fori_loop(0, _ROWS, fill_row, 0)
    tgt_cp.wait()

    def poke_grp(c, buf, g, value_vec):
        t = tgt_v[pl.ds(c * _COLW + g * _LANES, _LANES)]
        mask = (t >= r0) & (t < r0 + _ROWS)
        plsc.store_scatter(buf, [t - r0, lane_iota + g * _LANES], value_vec,
                           mask=mask)

    def scatter_chunk(c, buf, value_vec):
        def grp(g, _):
            poke_grp(c, buf, g, value_vec)
            return 0
        lax.fori_loop(0, _GROUPS, grp, 0)

    def restore_and_poke(c, buf):
        def grp(g, _):
            poke_grp(c - _NBUF, buf, g, base_vec)
            poke_grp(c, buf, g, peak_vec)
            return 0
        lax.fori_loop(0, _GROUPS, grp, 0)

    copies = [None] * _NBUF
    for c in range(_NCHUNKS):
        slot = c % _NBUF
        buf = bufs[slot]
        if copies[slot] is not None:
            copies[slot].wait()
            restore_and_poke(c, buf)
        else:
            scatter_chunk(c, buf, peak_vec)
        copies[slot] = pltpu.async_copy(
            buf, out_hbm.at[pl.ds(r0, _ROWS), pl.ds(c * _COLW, _COLW)],
            sems[slot])

    for slot in range(_NBUF):
        copies[slot].wait()


@jax.jit
def _sc_call(target):
    mesh = plsc.VectorSubcoreMesh(core_axis_name="c", subcore_axis_name="s")
    q_t = pl.kernel(
        _sc_body,
        mesh=mesh,
        compiler_params=pltpu.CompilerParams(needs_layout_passes=False),
        out_type=jax.ShapeDtypeStruct((_NUM_CLASSES, _BATCH), jnp.float32),
        scratch_types=[
            pltpu.VMEM((_BATCH,), jnp.int32),
        ] + [pltpu.VMEM((_ROWS, _COLW), jnp.float32)] * _NBUF
          + [pltpu.SemaphoreType.DMA] * (_NBUF + 1),
    )(target)
    return q_t.T


def kernel(target, pred):
    del pred
    return _sc_call(target)


# speedup vs baseline: 1.0831x; 1.0034x over previous
import jax
import jax.numpy as jnp
import numpy as np
from jax import lax
from jax.experimental import pallas as pl
from jax.experimental.pallas import tpu as pltpu
from jax.experimental.pallas import tpu_sc as plsc

_NUM_CLASSES = 1000
_SMOOTHING = 0.1
_BATCH = 16384

_NUM_WORKERS = 32
_ROWS = 32
_COLW = 512
_NCHUNKS = _BATCH // _COLW
_NBUF = 6
_LANES = 16
_GROUPS = _COLW // _LANES

_BASE = float(np.float32(_SMOOTHING / _NUM_CLASSES))
_PEAK = float(np.float32(np.float32(_BASE) + np.float32(1.0 - _SMOOTHING)))


def _sc_body(target_hbm, out_hbm, tgt_v, *rest):
    bufs = rest[:_NBUF]
    sems = rest[_NBUF:2 * _NBUF]
    tgt_sem = rest[2 * _NBUF]
    wid = lax.axis_index("s") * 2 + lax.axis_index("c")
    r0 = jnp.minimum(wid * _ROWS, _NUM_CLASSES - _ROWS)

    base_vec = jnp.full((_LANES,), _BASE, jnp.float32)
    peak_vec = jnp.full((_LANES,), _PEAK, jnp.float32)
    lane_iota = lax.broadcasted_iota(jnp.int32, (_LANES,), 0)

    tgt_cp = pltpu.async_copy(target_hbm, tgt_v, tgt_sem)

    def fill_row(r, _):
        def fill_grp(g, _):
            for b in bufs:
                b[r, pl.ds(g * _LANES, _LANES)] = base_vec
            return 0
        lax.fori_loop(0, _GROUPS, fill_grp, 0)
        return 0

    lax.fori_loop(0, _ROWS, fill_row, 0)
    tgt_cp.wait()

    def poke_grp(c, buf, g, value_vec):
        t = tgt_v[pl.ds(c * _COLW + g * _LANES, _LANES)]
        rows = t - r0
        mask = plsc.bitcast(rows, jnp.uint32) < jnp.uint32(_ROWS)
        plsc.store_scatter(buf, [rows, lane_iota + g * _LANES], value_vec,
                           mask=mask)

    def scatter_chunk(c, buf, value_vec):
        def grp(g, _):
            poke_grp(c, buf, g, value_vec)
            return 0
        lax.fori_loop(0, _GROUPS, grp, 0)

    def restore_and_poke(c, buf):
        def grp(g, _):
            poke_grp(c - _NBUF, buf, g, base_vec)
            poke_grp(c, buf, g, peak_vec)
            return 0
        lax.fori_loop(0, _GROUPS, grp, 0)

    copies = [None] * _NBUF
    for c in range(_NCHUNKS):
        slot = c % _NBUF
        buf = bufs[slot]
        if copies[slot] is not None:
            copies[slot].wait()
            restore_and_poke(c, buf)
        else:
            scatter_chunk(c, buf, peak_vec)
        copies[slot] = pltpu.async_copy(
            buf, out_hbm.at[pl.ds(r0, _ROWS), pl.ds(c * _COLW, _COLW)],
            sems[slot])

    for slot in range(_NBUF):
        copies[slot].wait()


@jax.jit
def _sc_call(target):
    mesh = plsc.VectorSubcoreMesh(core_axis_name="c", subcore_axis_name="s")
    q_t = pl.kernel(
        _sc_body,
        mesh=mesh,
        compiler_params=pltpu.CompilerParams(needs_layout_passes=False),
        out_type=jax.ShapeDtypeStruct((_NUM_CLASSES, _BATCH), jnp.float32),
        scratch_types=[
            pltpu.VMEM((_BATCH,), jnp.int32),
        ] + [pltpu.VMEM((_ROWS, _COLW), jnp.float32)] * _NBUF
          + [pltpu.SemaphoreType.DMA] * (_NBUF + 1),
    )(target)
    return q_t.T


def kernel(target, pred):
    del pred
    return _sc_call(target)


# speedup vs baseline: 1.0995x; 1.0152x over previous
import jax
import jax.numpy as jnp
import numpy as np
from jax import lax
from jax.experimental import pallas as pl
from jax.experimental.pallas import tpu as pltpu
from jax.experimental.pallas import tpu_sc as plsc

_NUM_CLASSES = 1000
_SMOOTHING = 0.1
_BATCH = 16384

_NUM_WORKERS = 32
_ROWS = 32
_COLW = 512
_NCHUNKS = _BATCH // _COLW
_NBUF = 6
_LANES = 16
_GROUPS = _COLW // _LANES

_BASE = float(np.float32(_SMOOTHING / _NUM_CLASSES))
_PEAK = float(np.float32(np.float32(_BASE) + np.float32(1.0 - _SMOOTHING)))


def _sc_body(target_hbm, out_hbm, tgt_v, *rest):
    bufs = rest[:_NBUF]
    sems = rest[_NBUF:2 * _NBUF]
    tgt_sem = rest[2 * _NBUF]
    wid = lax.axis_index("s") * 2 + lax.axis_index("c")
    r0 = jnp.minimum(wid * _ROWS, _NUM_CLASSES - _ROWS)

    base_vec = jnp.full((_LANES,), _BASE, jnp.float32)
    peak_vec = jnp.full((_LANES,), _PEAK, jnp.float32)
    lane_iota = lax.broadcasted_iota(jnp.int32, (_LANES,), 0)

    tgt_cp = pltpu.async_copy(target_hbm, tgt_v, tgt_sem)

    def fill_row(r, _):
        def fill_grp(g, _):
            for b in bufs:
                b[r, pl.ds(g * _LANES, _LANES)] = base_vec
            return 0
        lax.fori_loop(0, _GROUPS, fill_grp, 0)
        return 0

    lax.fori_loop(0, _ROWS, fill_row, 0)
    tgt_cp.wait()

    def poke_grp(c, buf, g, value_vec):
        t = tgt_v[pl.ds(c * _COLW + g * _LANES, _LANES)]
        rows = t - r0
        mask = plsc.bitcast(rows, jnp.uint32) < jnp.uint32(_ROWS)
        plsc.store_scatter(buf, [rows, lane_iota + g * _LANES], value_vec,
                           mask=mask)

    def scatter_chunk(c, buf, value_vec):
        def grp(g, _):
            poke_grp(c, buf, g, value_vec)
            return 0
        lax.fori_loop(0, _GROUPS, grp, 0)

    def restore_and_poke(c, buf):
        def grp(g, _):
            poke_grp(c - _NBUF, buf, g, base_vec)
            poke_grp(c, buf, g, peak_vec)
            return 0
        lax.fori_loop(0, _GROUPS, grp, 0)

    def fire(c, slot):
        pltpu.async_copy(
            bufs[slot], out_hbm.at[pl.ds(r0, _ROWS), pl.ds(c * _COLW, _COLW)],
            sems[slot])

    def drain(slot):
        pltpu.make_async_copy(
            bufs[slot], out_hbm.at[pl.ds(r0, _ROWS), pl.ds(0, _COLW)],
            sems[slot]).wait()

    for c in range(_NBUF):
        scatter_chunk(c, bufs[c], peak_vec)
        fire(c, c)

    _ROUNDS = (_NCHUNKS - _NBUF) // _NBUF

    def round_body(r, _):
        for b in range(_NBUF):
            c = _NBUF + r * _NBUF + b
            drain(b)
            restore_and_poke(c, bufs[b])
            fire(c, b)
        return 0

    lax.fori_loop(0, _ROUNDS, round_body, 0)

    for c in range(_NBUF + _ROUNDS * _NBUF, _NCHUNKS):
        slot = c % _NBUF
        drain(slot)
        restore_and_poke(c, bufs[slot])
        fire(c, slot)

    for slot in range(_NBUF):
        drain(slot)


@jax.jit
def _sc_call(target):
    mesh = plsc.VectorSubcoreMesh(core_axis_name="c", subcore_axis_name="s")
    q_t = pl.kernel(
        _sc_body,
        mesh=mesh,
        compiler_params=pltpu.CompilerParams(needs_layout_passes=False),
        out_type=jax.ShapeDtypeStruct((_NUM_CLASSES, _BATCH), jnp.float32),
        scratch_types=[
            pltpu.VMEM((_BATCH,), jnp.int32),
        ] + [pltpu.VMEM((_ROWS, _COLW), jnp.float32)] * _NBUF
          + [pltpu.SemaphoreType.DMA] * (_NBUF + 1),
    )(target)
    return q_t.T


def kernel(target, pred):
    del pred
    return _sc_call(target)
